# baseline (speedup 1.0000x reference)
import jax
import jax.numpy as jnp
from jax import lax
from jax.experimental import pallas as pl
from jax.experimental.pallas import tpu as pltpu
from jax.experimental.pallas import tpu_sc as plsc

EMB = 64
BATCH = 16384
NC = 2
NS = 16
NW = NC * NS
PER_W = BATCH // NW
CHUNK = 128
NCHUNK = PER_W // CHUNK
GROUPS = CHUNK // 16


def _body(srow_hbm, orow_hbm, rrow_hbm, trow_hbm,
          rcol_hbm, tcol_hbm,
          e_hbm, re_hbm, te_hbm, out_hbm,
          s_idx, o_idx, r_idx, t_idx,
          r_col, t_col,
          sb, ob, rb, tb, res,
          sem_s, sem_o, sem_r, sem_t):
    wid = lax.axis_index("s") * NC + lax.axis_index("c")

    for ch in range(NCHUNK):
        row = wid * NCHUNK + ch
        pltpu.sync_copy(srow_hbm.at[row], s_idx.at[ch])
        pltpu.sync_copy(orow_hbm.at[row], o_idx.at[ch])
        pltpu.sync_copy(rrow_hbm.at[row], r_idx.at[ch])
        pltpu.sync_copy(trow_hbm.at[row], t_idx.at[ch])
        pltpu.sync_copy(rcol_hbm.at[row], r_col.at[ch])
        pltpu.sync_copy(tcol_hbm.at[row], t_col.at[ch])

    iota = lax.iota(jnp.int32, 16)

    for ch in range(NCHUNK):
        cs = pltpu.async_copy(e_hbm.at[s_idx.at[ch]], sb, sem_s)
        co = pltpu.async_copy(e_hbm.at[o_idx.at[ch]], ob, sem_o)
        cr = pltpu.async_copy(re_hbm.at[r_idx.at[ch]], rb, sem_r)
        ct = pltpu.async_copy(te_hbm.at[t_idx.at[ch]], tb, sem_t)
        cs.wait()
        co.wait()
        cr.wait()
        ct.wait()

        for g in range(GROUPS):
            rid = iota + (g * 16)
            br = r_col[ch, pl.ds(g * 16, 16)]
            bt = t_col[ch, pl.ds(g * 16, 16)]

            def col_body(j, carry, br=br, bt=bt, rid=rid):
                acc, col = carry
                vs = plsc.load_gather(sb, [rid, col])
                vr = plsc.load_gather(rb, [rid, br + col])
                vt = plsc.load_gather(tb, [rid, bt + col])
                vo = plsc.load_gather(ob, [rid, col])
                return acc + jnp.abs(vs + vr + vt - vo), (col + 1) & 63

            (acc, _) = plsc.parallel_loop(
                0, EMB, carry=(jnp.zeros((16,), jnp.float32), iota),
                unroll=8)(col_body)
            res[pl.ds(ch * CHUNK + g * 16, 16)] = acc

    pltpu.sync_copy(res, out_hbm.at[pl.ds(wid * PER_W, PER_W)])


def _pad_body(x_ref, o_ref):
    x = x_ref[...]
    o_ref[...] = jnp.concatenate([x, x], axis=1)


_PAD_BR = 8000


def _pad_view(e_embed):
    n = e_embed.shape[0]
    return pl.pallas_call(
        _pad_body,
        grid=(n // _PAD_BR,),
        in_specs=[pl.BlockSpec((_PAD_BR, EMB), lambda i: (i, 0))],
        out_specs=pl.BlockSpec((_PAD_BR, 2 * EMB), lambda i: (i, 0)),
        out_shape=jax.ShapeDtypeStruct((n, 2 * EMB), jnp.float32),
    )(e_embed)


@jax.jit
def _run(s, o, r, t, e_embed, r_embed, t_embed):
    si = s.astype(jnp.int32)
    oi = o.astype(jnp.int32)
    ri = r.astype(jnp.int32)
    ti = t.astype(jnp.int32)

    def chunked(x):
        return x.reshape(NW * NCHUNK, CHUNK)

    e2 = _pad_view(e_embed)
    re2 = r_embed.reshape(-1, 2 * EMB)
    te2 = t_embed.reshape(-1, 2 * EMB)

    mesh = plsc.VectorSubcoreMesh(core_axis_name="c", subcore_axis_name="s")
    run = pl.kernel(
        _body,
        out_type=jax.ShapeDtypeStruct((BATCH,), jnp.float32),
        mesh=mesh,
        compiler_params=pltpu.CompilerParams(
            needs_layout_passes=False, use_tc_tiling_on_sc=True),
        scratch_types=[
            pltpu.VMEM((NCHUNK, CHUNK), jnp.int32),
            pltpu.VMEM((NCHUNK, CHUNK), jnp.int32),
            pltpu.VMEM((NCHUNK, CHUNK), jnp.int32),
            pltpu.VMEM((NCHUNK, CHUNK), jnp.int32),
            pltpu.VMEM((NCHUNK, CHUNK), jnp.int32),
            pltpu.VMEM((NCHUNK, CHUNK), jnp.int32),
            pltpu.VMEM((CHUNK, 2 * EMB), jnp.float32),
            pltpu.VMEM((CHUNK, 2 * EMB), jnp.float32),
            pltpu.VMEM((CHUNK, 2 * EMB), jnp.float32),
            pltpu.VMEM((CHUNK, 2 * EMB), jnp.float32),
            pltpu.VMEM((PER_W,), jnp.float32),
            pltpu.SemaphoreType.DMA,
            pltpu.SemaphoreType.DMA,
            pltpu.SemaphoreType.DMA,
            pltpu.SemaphoreType.DMA,
        ],
    )
    return run(chunked(si), chunked(oi),
               chunked(ri >> 1), chunked(ti >> 1),
               chunked((ri & 1) * EMB), chunked((ti & 1) * EMB),
               e2, re2, te2)


def kernel(s, o, r, t, e_embed, r_embed, t_embed):
    return _run(s, o, r, t, e_embed, r_embed, t_embed)


# speedup vs baseline: 1.2238x; 1.2238x over previous
import jax
import jax.numpy as jnp
from jax import lax
from jax.experimental import pallas as pl
from jax.experimental.pallas import tpu as pltpu
from jax.experimental.pallas import tpu_sc as plsc

EMB = 64
BATCH = 16384
NC = 2
NS = 16
NW = NC * NS
PER_W = BATCH // NW
CHUNK = 128
NCHUNK = PER_W // CHUNK
GROUPS = CHUNK // 16


def _body(srow_hbm, orow_hbm, rrow_hbm, trow_hbm,
          rcol_hbm, tcol_hbm,
          e_hbm, re_hbm, te_hbm, out_hbm,
          s_idx, o_idx, r_idx, t_idx,
          r_col, t_col,
          sb, ob, rb, tb, res,
          sem_s, sem_o, sem_r, sem_t):
    wid = lax.axis_index("s") * NC + lax.axis_index("c")

    for ch in range(NCHUNK):
        row = wid * NCHUNK + ch
        pltpu.sync_copy(srow_hbm.at[row], s_idx.at[ch])
        pltpu.sync_copy(orow_hbm.at[row], o_idx.at[ch])
        pltpu.sync_copy(rrow_hbm.at[row], r_idx.at[ch])
        pltpu.sync_copy(trow_hbm.at[row], t_idx.at[ch])
        pltpu.sync_copy(rcol_hbm.at[row], r_col.at[ch])
        pltpu.sync_copy(tcol_hbm.at[row], t_col.at[ch])

    iota = lax.iota(jnp.int32, 16)

    for ch in range(NCHUNK):
        cs = pltpu.async_copy(e_hbm.at[s_idx.at[ch]], sb, sem_s)
        co = pltpu.async_copy(e_hbm.at[o_idx.at[ch]], ob, sem_o)
        cr = pltpu.async_copy(re_hbm.at[r_idx.at[ch]], rb, sem_r)
        ct = pltpu.async_copy(te_hbm.at[t_idx.at[ch]], tb, sem_t)
        cs.wait()
        co.wait()
        cr.wait()
        ct.wait()

        for g in range(GROUPS):
            rid = iota + (g * 16)
            br = r_col[ch, pl.ds(g * 16, 16)]
            bt = t_col[ch, pl.ds(g * 16, 16)]

            def col_body(j, carry, br=br, bt=bt, rid=rid):
                acc, col = carry
                vs = plsc.load_gather(sb, [rid, col])
                vr = plsc.load_gather(rb, [rid, br + col])
                vt = plsc.load_gather(tb, [rid, bt + col])
                vo = plsc.load_gather(ob, [rid, col])
                return acc + jnp.abs(vs + vr + vt - vo), (col + 1) & 63

            (acc, _) = plsc.parallel_loop(
                0, EMB, carry=(jnp.zeros((16,), jnp.float32), iota),
                unroll=8)(col_body)
            res[pl.ds(ch * CHUNK + g * 16, 16)] = acc

    pltpu.sync_copy(res, out_hbm.at[pl.ds(wid * PER_W, PER_W)])


def _pad_body(x_ref, o_ref):
    x = x_ref[...]
    o_ref[...] = jnp.concatenate([x, x], axis=1)


_PAD_BR = 8000


def _pad_view(e_embed):
    n = e_embed.shape[0]
    return pl.pallas_call(
        _pad_body,
        grid=(n // _PAD_BR,),
        in_specs=[pl.BlockSpec((_PAD_BR, EMB), lambda i: (i, 0))],
        out_specs=pl.BlockSpec((_PAD_BR, 2 * EMB), lambda i: (i, 0)),
        out_shape=jax.ShapeDtypeStruct((n, 2 * EMB), jnp.float32),
    )(e_embed)


@jax.jit
def _run(s, o, r, t, e_embed, r_embed, t_embed):
    si = s.astype(jnp.int32)
    oi = o.astype(jnp.int32)
    ri = r.astype(jnp.int32)
    ti = t.astype(jnp.int32)

    def chunked(x):
        return x.reshape(NW * NCHUNK, CHUNK)

    e2 = jnp.pad(e_embed, ((0, 0), (0, EMB)))
    re2 = r_embed.reshape(-1, 2 * EMB)
    te2 = t_embed.reshape(-1, 2 * EMB)

    mesh = plsc.VectorSubcoreMesh(core_axis_name="c", subcore_axis_name="s")
    run = pl.kernel(
        _body,
        out_type=jax.ShapeDtypeStruct((BATCH,), jnp.float32),
        mesh=mesh,
        compiler_params=pltpu.CompilerParams(
            needs_layout_passes=False, use_tc_tiling_on_sc=True),
        scratch_types=[
            pltpu.VMEM((NCHUNK, CHUNK), jnp.int32),
            pltpu.VMEM((NCHUNK, CHUNK), jnp.int32),
            pltpu.VMEM((NCHUNK, CHUNK), jnp.int32),
            pltpu.VMEM((NCHUNK, CHUNK), jnp.int32),
            pltpu.VMEM((NCHUNK, CHUNK), jnp.int32),
            pltpu.VMEM((NCHUNK, CHUNK), jnp.int32),
            pltpu.VMEM((CHUNK, 2 * EMB), jnp.float32),
            pltpu.VMEM((CHUNK, 2 * EMB), jnp.float32),
            pltpu.VMEM((CHUNK, 2 * EMB), jnp.float32),
            pltpu.VMEM((CHUNK, 2 * EMB), jnp.float32),
            pltpu.VMEM((PER_W,), jnp.float32),
            pltpu.SemaphoreType.DMA,
            pltpu.SemaphoreType.DMA,
            pltpu.SemaphoreType.DMA,
            pltpu.SemaphoreType.DMA,
        ],
    )
    return run(chunked(si), chunked(oi),
               chunked(ri >> 1), chunked(ti >> 1),
               chunked((ri & 1) * EMB), chunked((ti & 1) * EMB),
               e2, re2, te2)


def kernel(s, o, r, t, e_embed, r_embed, t_embed):
    return _run(s, o, r, t, e_embed, r_embed, t_embed)
